# Initial kernel scaffold; baseline (speedup 1.0000x reference)
import jax
import jax.numpy as jnp
from jax.experimental import pallas as pl


def kernel(x, table, W, b):
    raise NotImplementedError("write your pallas kernel here")


# speedup vs baseline: 1.1777x; 1.1777x over previous
import functools

import jax
import jax.numpy as jnp
from jax import lax
from jax.experimental import pallas as pl
from jax.experimental.pallas import tpu as pltpu
from jax.experimental.pallas import tpu_sc as plsc

B, L = 16384, 50
VOCAB, D = 1000000, 64
R = B * L
NC, NS = 2, 16
NW = NC * NS
IDX_COLS = 128
ROWS_W = R // (NW * IDX_COLS)
MM_BLK = 8000


def _mm_body(t_ref, w_ref, b_ref, o_ref):
    o_ref[...] = (
        jnp.dot(t_ref[...], w_ref[...], preferred_element_type=jnp.float32)
        + b_ref[...]
    )


def _table_times_w(table, W, b):
    return pl.pallas_call(
        _mm_body,
        grid=(VOCAB // MM_BLK,),
        in_specs=[
            pl.BlockSpec((MM_BLK, D), lambda i: (i, 0)),
            pl.BlockSpec((D, D), lambda i: (0, 0)),
            pl.BlockSpec((1, D), lambda i: (0, 0)),
        ],
        out_specs=pl.BlockSpec((MM_BLK, D), lambda i: (i, 0)),
        out_shape=jax.ShapeDtypeStruct((VOCAB, D), jnp.float32),
    )(table, W, b.reshape(1, D))


def _gather_body(x_hbm, tw_hbm, out_hbm, mask_hbm,
                 idx_v, buf0, buf1, mask_v, sem0, sem1):
    wid = lax.axis_index("s") * NC + lax.axis_index("c")
    row0 = wid * ROWS_W
    out0 = row0 * IDX_COLS

    pltpu.sync_copy(x_hbm.at[pl.ds(row0, ROWS_W)], idx_v)

    def start(j, buf, sem):
        pltpu.make_async_copy(tw_hbm.at[idx_v.at[j]], buf, sem).start()

    def finish(j, buf, sem):
        pltpu.make_async_copy(tw_hbm.at[idx_v.at[j]], buf, sem).wait()
        pltpu.sync_copy(buf, out_hbm.at[pl.ds(out0 + j * IDX_COLS, IDX_COLS)])
        for k in range(IDX_COLS // 16):
            v = idx_v[j, pl.ds(k * 16, 16)]
            mask_v[j, pl.ds(k * 16, 16)] = (
                1 - jnp.minimum(v, 1)).astype(jnp.float32)

    start(0, buf0, sem0)

    def pair(j2, carry):
        j0 = 2 * j2
        start(j0 + 1, buf1, sem1)
        finish(j0, buf0, sem0)

        @pl.when(j2 + 1 < ROWS_W // 2)
        def _():
            start(j0 + 2, buf0, sem0)

        finish(j0 + 1, buf1, sem1)
        return carry

    lax.fori_loop(0, ROWS_W // 2, pair, 0)
    pltpu.sync_copy(mask_v, mask_hbm.at[pl.ds(row0, ROWS_W)])


@functools.lru_cache(maxsize=1)
def _gather_kernel():
    return pl.kernel(
        _gather_body,
        out_type=(
            jax.ShapeDtypeStruct((R, D), jnp.float32),
            jax.ShapeDtypeStruct((R // IDX_COLS, IDX_COLS), jnp.float32),
        ),
        mesh=plsc.VectorSubcoreMesh(
            core_axis_name="c", subcore_axis_name="s",
            num_cores=NC, num_subcores=NS,
        ),
        scratch_types=[
            pltpu.VMEM((ROWS_W, IDX_COLS), jnp.int32),
            pltpu.VMEM((IDX_COLS, D), jnp.float32),
            pltpu.VMEM((IDX_COLS, D), jnp.float32),
            pltpu.VMEM((ROWS_W, IDX_COLS), jnp.float32),
            pltpu.SemaphoreType.DMA,
            pltpu.SemaphoreType.DMA,
        ],
        compiler_params=pltpu.CompilerParams(use_tc_tiling_on_sc=False),
    )


def _emb_times_w(emb, W, b):
    return pl.pallas_call(
        _mm_body,
        grid=(R // 8192,),
        in_specs=[
            pl.BlockSpec((8192, D), lambda i: (i, 0)),
            pl.BlockSpec((D, D), lambda i: (0, 0)),
            pl.BlockSpec((1, D), lambda i: (0, 0)),
        ],
        out_specs=pl.BlockSpec((8192, D), lambda i: (i, 0)),
        out_shape=jax.ShapeDtypeStruct((R, D), jnp.float32),
    )(emb, W, b.reshape(1, D))


def kernel(x, table, W, b):
    x2d = x.astype(jnp.int32).reshape(R // IDX_COLS, IDX_COLS)
    emb_flat, mask2d = _gather_kernel()(x2d, table)
    out_flat = _emb_times_w(emb_flat, W, b)
    return out_flat.reshape(B, L, D), mask2d.reshape(B, L)


# speedup vs baseline: 1.4533x; 1.2340x over previous
import functools

import jax
import jax.numpy as jnp
from jax import lax
from jax.experimental import pallas as pl
from jax.experimental.pallas import tpu as pltpu
from jax.experimental.pallas import tpu_sc as plsc

B, L = 16384, 50
VOCAB, D = 1000000, 64
R = B * L
NC, NS = 2, 16
NW = NC * NS
IDX_COLS = 128
ROWS_W = R // (NW * IDX_COLS)
MM_BLK = 8000


def _mm_body(t_ref, w_ref, b_ref, o_ref):
    tw = (
        jnp.dot(t_ref[...], w_ref[...], preferred_element_type=jnp.float32)
        + b_ref[...]
    )
    o_ref[:, 0:D] = tw
    o_ref[:, D:2 * D] = tw


def _table_times_w(table, W, b):
    return pl.pallas_call(
        _mm_body,
        grid=(VOCAB // MM_BLK,),
        in_specs=[
            pl.BlockSpec((MM_BLK, D), lambda i: (i, 0)),
            pl.BlockSpec((D, D), lambda i: (0, 0)),
            pl.BlockSpec((1, D), lambda i: (0, 0)),
        ],
        out_specs=pl.BlockSpec((MM_BLK, 2 * D), lambda i: (i, 0)),
        out_shape=jax.ShapeDtypeStruct((VOCAB, 2 * D), jnp.float32),
    )(table, W, b.reshape(1, D))


def _gather_body(x_hbm, tw_hbm, out_hbm, mask_hbm,
                 idx_v, buf0, buf1, mask_v, sem0, sem1):
    wid = lax.axis_index("s") * NC + lax.axis_index("c")
    row0 = wid * ROWS_W
    out0 = row0 * IDX_COLS

    pltpu.sync_copy(x_hbm.at[pl.ds(row0, ROWS_W)], idx_v)

    def start(j, buf, sem):
        pltpu.make_async_copy(tw_hbm.at[idx_v.at[j]], buf, sem).start()

    def finish(j, buf, sem):
        pltpu.make_async_copy(tw_hbm.at[idx_v.at[j]], buf, sem).wait()
        pltpu.sync_copy(buf, out_hbm.at[pl.ds(out0 + j * IDX_COLS, IDX_COLS)])
        for k in range(IDX_COLS // 16):
            v = idx_v[j, pl.ds(k * 16, 16)]
            mask_v[j, pl.ds(k * 16, 16)] = (
                1 - jnp.minimum(v, 1)).astype(jnp.float32)

    start(0, buf0, sem0)

    def pair(j2, carry):
        j0 = 2 * j2
        start(j0 + 1, buf1, sem1)
        finish(j0, buf0, sem0)

        @pl.when(j2 + 1 < ROWS_W // 2)
        def _():
            start(j0 + 2, buf0, sem0)

        finish(j0 + 1, buf1, sem1)
        return carry

    lax.fori_loop(0, ROWS_W // 2, pair, 0)
    pltpu.sync_copy(mask_v, mask_hbm.at[pl.ds(row0, ROWS_W)])


@functools.lru_cache(maxsize=1)
def _gather_kernel():
    return pl.kernel(
        _gather_body,
        out_type=(
            jax.ShapeDtypeStruct((R, 2 * D), jnp.float32),
            jax.ShapeDtypeStruct((R // IDX_COLS, IDX_COLS), jnp.float32),
        ),
        mesh=plsc.VectorSubcoreMesh(
            core_axis_name="c", subcore_axis_name="s",
            num_cores=NC, num_subcores=NS,
        ),
        scratch_types=[
            pltpu.VMEM((ROWS_W, IDX_COLS), jnp.int32),
            pltpu.VMEM((IDX_COLS, 2 * D), jnp.float32),
            pltpu.VMEM((IDX_COLS, 2 * D), jnp.float32),
            pltpu.VMEM((ROWS_W, IDX_COLS), jnp.float32),
            pltpu.SemaphoreType.DMA,
            pltpu.SemaphoreType.DMA,
        ],
        compiler_params=pltpu.CompilerParams(use_tc_tiling_on_sc=True),
    )


def kernel(x, table, W, b):
    tw = _table_times_w(table, W, b)
    x2d = x.astype(jnp.int32).reshape(R // IDX_COLS, IDX_COLS)
    out128, mask2d = _gather_kernel()(x2d, tw)
    return out128[:, :D].reshape(B, L, D), mask2d.reshape(B, L)
